# Initial kernel scaffold; baseline (speedup 1.0000x reference)
import functools

import jax
import jax.numpy as jnp
from jax import lax
from jax.experimental import pallas as pl
from jax.experimental.pallas import tpu as pltpu
from jax.experimental.pallas import tpu_sc as plsc

NUM_CORES = 2
NUM_SUBCORES = 16
LANES = 16
NW = NUM_CORES * NUM_SUBCORES

ROWS = 128


def _body(x_hbm, locs_hbm, coeffs_hbm, out_hbm,
          locs_v, coeffs_v, xb0, xb1, ob0, ob1, is0, is1, os0, os1):
  B, A = x_hbm.shape
  S = locs_hbm.shape[1]
  cpw = A // NW
  n_groups = cpw // LANES
  n_chunks = B // ROWS

  wid = lax.axis_index("s") * NUM_CORES + lax.axis_index("c")
  c0 = wid * cpw

  pltpu.sync_copy(locs_hbm.at[pl.ds(c0, cpw)], locs_v)
  pltpu.sync_copy(coeffs_hbm.at[pl.ds(c0, cpw)], coeffs_v)

  bufs = ((xb0, ob0, is0, os0), (xb1, ob1, is1, os1))

  def in_copy(g, xb, sem):
    return pltpu.make_async_copy(
        x_hbm.at[pl.ds(g * ROWS, ROWS), pl.ds(c0, cpw)], xb, sem)

  def out_copy(g, ob, sem):
    return pltpu.make_async_copy(
        ob, out_hbm.at[pl.ds(g * ROWS, ROWS), pl.ds(c0, cpw)], sem)

  iota = lax.iota(jnp.int32, LANES)

  def do_chunk(xb, ob):
    def row_body(r, carry):
      for cg in range(n_groups):
        chl = iota + (cg * LANES)
        xv = xb[r, pl.ds(cg * LANES, LANES)]
        lo = jnp.zeros((LANES,), jnp.int32)
        step = S // 2
        while step >= 1:
          v = plsc.load_gather(locs_v, [chl, lo + (step - 1)])
          lo = jnp.where(v < xv, lo + step, lo)
          step //= 2
        idx = jnp.maximum(lo - 1, 0)
        idx1 = idx + 1
        x0 = plsc.load_gather(locs_v, [chl, idx])
        x1 = plsc.load_gather(locs_v, [chl, idx1])
        y0 = plsc.load_gather(coeffs_v, [chl, idx])
        y1 = plsc.load_gather(coeffs_v, [chl, idx1])
        t = (xv - x0) / (x1 - x0 + 1e-6)
        ob[r, pl.ds(cg * LANES, LANES)] = y0 + t * (y1 - y0)
      return carry
    lax.fori_loop(0, ROWS, row_body, 0)

  in_copy(0, xb0, is0).start()
  in_copy(1, xb1, is1).start()

  def outer(g2, carry):
    for b, (xb, ob, isem, osem) in enumerate(bufs):
      g = g2 * 2 + b
      in_copy(g, xb, isem).wait()

      @pl.when(g2 >= 1)
      def _wait_prev_out():
        out_copy(g - 2, ob, osem).wait()

      do_chunk(xb, ob)
      out_copy(g, ob, osem).start()

      @pl.when(g2 < n_chunks // 2 - 1)
      def _start_next_in():
        in_copy(g + 2, xb, isem).start()
    return carry

  lax.fori_loop(0, n_chunks // 2, outer, 0)

  out_copy(n_chunks - 2, ob0, os0).wait()
  out_copy(n_chunks - 1, ob1, os1).wait()


def kernel(x, locs, coeffs):
  B, A = x.shape
  S = locs.shape[1]
  cpw = A // NW
  mesh = plsc.VectorSubcoreMesh(core_axis_name="c", subcore_axis_name="s")
  f = functools.partial(
      pl.kernel,
      out_type=jax.ShapeDtypeStruct((B, A), jnp.float32),
      mesh=mesh,
      scratch_types=[
          pltpu.VMEM((cpw, S), jnp.float32),
          pltpu.VMEM((cpw, S), jnp.float32),
          pltpu.VMEM((ROWS, cpw), jnp.float32),
          pltpu.VMEM((ROWS, cpw), jnp.float32),
          pltpu.VMEM((ROWS, cpw), jnp.float32),
          pltpu.VMEM((ROWS, cpw), jnp.float32),
          pltpu.SemaphoreType.DMA,
          pltpu.SemaphoreType.DMA,
          pltpu.SemaphoreType.DMA,
          pltpu.SemaphoreType.DMA,
      ],
  )(_body)
  return f(x, locs, coeffs)


# speedup vs baseline: 1667.0870x; 1667.0870x over previous
import functools

import jax
import jax.numpy as jnp
from jax import lax
from jax.experimental import pallas as pl
from jax.experimental.pallas import tpu as pltpu
from jax.experimental.pallas import tpu_sc as plsc

NUM_CORES = 2
NUM_SUBCORES = 16
LANES = 16
NW = NUM_CORES * NUM_SUBCORES

ROWS = 128


def _body(x_hbm, locs_hbm, coeffs_hbm, out_hbm,
          locs_v, coeffs_v, xb0, xb1, ob0, ob1, is0, is1, os0, os1):
  B, A = x_hbm.shape
  S = locs_hbm.shape[0] // A
  cpw = A // NW
  n_groups = cpw // LANES
  n_chunks = B // ROWS

  wid = lax.axis_index("s") * NUM_CORES + lax.axis_index("c")
  c0 = wid * cpw

  pltpu.sync_copy(locs_hbm.at[pl.ds(c0 * S, cpw * S)], locs_v)
  pltpu.sync_copy(coeffs_hbm.at[pl.ds(c0 * S, cpw * S)], coeffs_v)

  bufs = ((xb0, ob0, is0, os0), (xb1, ob1, is1, os1))

  def in_copy(g, xb, sem):
    return pltpu.make_async_copy(
        x_hbm.at[pl.ds(g * ROWS, ROWS), pl.ds(c0, cpw)], xb, sem)

  def out_copy(g, ob, sem):
    return pltpu.make_async_copy(
        ob, out_hbm.at[pl.ds(g * ROWS, ROWS), pl.ds(c0, cpw)], sem)

  iota = lax.iota(jnp.int32, LANES)

  def do_chunk(xb, ob):
    def row_body(r, carry):
      for cg in range(n_groups):
        base = (iota + (cg * LANES)) * S
        xv = xb[r, pl.ds(cg * LANES, LANES)]
        lo = jnp.zeros((LANES,), jnp.int32)
        step = S // 2
        while step >= 1:
          v = plsc.load_gather(locs_v, [base + (lo + (step - 1))])
          lo = jnp.where(v < xv, lo + step, lo)
          step //= 2
        idx = jnp.maximum(lo - 1, 0)
        idx1 = idx + 1
        x0 = plsc.load_gather(locs_v, [base + idx])
        x1 = plsc.load_gather(locs_v, [base + idx1])
        y0 = plsc.load_gather(coeffs_v, [base + idx])
        y1 = plsc.load_gather(coeffs_v, [base + idx1])
        t = (xv - x0) / (x1 - x0 + 1e-6)
        ob[r, pl.ds(cg * LANES, LANES)] = y0 + t * (y1 - y0)
      return carry
    lax.fori_loop(0, ROWS, row_body, 0)

  in_copy(0, xb0, is0).start()
  in_copy(1, xb1, is1).start()

  def outer(g2, carry):
    for b, (xb, ob, isem, osem) in enumerate(bufs):
      g = g2 * 2 + b
      in_copy(g, xb, isem).wait()

      @pl.when(g2 >= 1)
      def _wait_prev_out():
        out_copy(g - 2, ob, osem).wait()

      do_chunk(xb, ob)
      out_copy(g, ob, osem).start()

      @pl.when(g2 < n_chunks // 2 - 1)
      def _start_next_in():
        in_copy(g + 2, xb, isem).start()
    return carry

  lax.fori_loop(0, n_chunks // 2, outer, 0)

  out_copy(n_chunks - 2, ob0, os0).wait()
  out_copy(n_chunks - 1, ob1, os1).wait()


def kernel(x, locs, coeffs):
  B, A = x.shape
  S = locs.shape[1]
  cpw = A // NW
  mesh = plsc.VectorSubcoreMesh(core_axis_name="c", subcore_axis_name="s")
  f = functools.partial(
      pl.kernel,
      out_type=jax.ShapeDtypeStruct((B, A), jnp.float32),
      mesh=mesh,
      compiler_params=pltpu.CompilerParams(needs_layout_passes=False),
      scratch_types=[
          pltpu.VMEM((cpw * S,), jnp.float32),
          pltpu.VMEM((cpw * S,), jnp.float32),
          pltpu.VMEM((ROWS, cpw), jnp.float32),
          pltpu.VMEM((ROWS, cpw), jnp.float32),
          pltpu.VMEM((ROWS, cpw), jnp.float32),
          pltpu.VMEM((ROWS, cpw), jnp.float32),
          pltpu.SemaphoreType.DMA,
          pltpu.SemaphoreType.DMA,
          pltpu.SemaphoreType.DMA,
          pltpu.SemaphoreType.DMA,
      ],
  )(_body)
  return f(x, locs.reshape(A * S), coeffs.reshape(A * S))


# speedup vs baseline: 4075.0845x; 2.4444x over previous
import functools

import jax
import jax.numpy as jnp
from jax import lax
from jax.experimental import pallas as pl
from jax.experimental.pallas import tpu as pltpu
from jax.experimental.pallas import tpu_sc as plsc

NUM_CORES = 2
NUM_SUBCORES = 16
LANES = 16
NW = NUM_CORES * NUM_SUBCORES

ROWS = 128


def _body(x_hbm, locs_hbm, coeffs_hbm, out_hbm,
          locs_v, coeffs_v, xb0, xb1, ob0, ob1, is0, is1, os0, os1):
  B, A = x_hbm.shape
  S = locs_hbm.shape[0] // A
  cpw = A // NW
  n_groups = cpw // LANES
  n_chunks = B // ROWS

  wid = lax.axis_index("s") * NUM_CORES + lax.axis_index("c")
  c0 = wid * cpw

  pltpu.sync_copy(locs_hbm.at[pl.ds(c0 * S, cpw * S)], locs_v)
  pltpu.sync_copy(coeffs_hbm.at[pl.ds(c0 * S, cpw * S)], coeffs_v)

  bufs = ((xb0, ob0, is0, os0), (xb1, ob1, is1, os1))

  def in_copy(g, xb, sem):
    return pltpu.make_async_copy(
        x_hbm.at[pl.ds(g * ROWS, ROWS), pl.ds(c0, cpw)], xb, sem)

  def out_copy(g, ob, sem):
    return pltpu.make_async_copy(
        ob, out_hbm.at[pl.ds(g * ROWS, ROWS), pl.ds(c0, cpw)], sem)

  iota = lax.iota(jnp.int32, LANES)
  G = range(n_groups)
  bases = [(iota + (cg * LANES)) * S for cg in G]

  def do_chunk(xb, ob):
    def row_body(r, carry):
      xv = [xb[r, pl.ds(cg * LANES, LANES)] for cg in G]
      lo = [jnp.zeros((LANES,), jnp.int32) for _ in G]
      step = S // 2
      while step >= 1:
        v = [plsc.load_gather(locs_v, [bases[cg] + (lo[cg] + (step - 1))])
             for cg in G]
        lo = [jnp.where(v[cg] < xv[cg], lo[cg] + step, lo[cg]) for cg in G]
        step //= 2
      idx = [bases[cg] + jnp.maximum(lo[cg] - 1, 0) for cg in G]
      x0 = [plsc.load_gather(locs_v, [idx[cg]]) for cg in G]
      x1 = [plsc.load_gather(locs_v, [idx[cg] + 1]) for cg in G]
      y0 = [plsc.load_gather(coeffs_v, [idx[cg]]) for cg in G]
      y1 = [plsc.load_gather(coeffs_v, [idx[cg] + 1]) for cg in G]
      for cg in G:
        t = (xv[cg] - x0[cg]) / (x1[cg] - x0[cg] + 1e-6)
        ob[r, pl.ds(cg * LANES, LANES)] = y0[cg] + t * (y1[cg] - y0[cg])
      return carry
    lax.fori_loop(0, ROWS, row_body, 0)

  in_copy(0, xb0, is0).start()
  in_copy(1, xb1, is1).start()

  def outer(g2, carry):
    for b, (xb, ob, isem, osem) in enumerate(bufs):
      g = g2 * 2 + b
      in_copy(g, xb, isem).wait()

      @pl.when(g2 >= 1)
      def _wait_prev_out():
        out_copy(g - 2, ob, osem).wait()

      do_chunk(xb, ob)
      out_copy(g, ob, osem).start()

      @pl.when(g2 < n_chunks // 2 - 1)
      def _start_next_in():
        in_copy(g + 2, xb, isem).start()
    return carry

  lax.fori_loop(0, n_chunks // 2, outer, 0)

  out_copy(n_chunks - 2, ob0, os0).wait()
  out_copy(n_chunks - 1, ob1, os1).wait()


def kernel(x, locs, coeffs):
  B, A = x.shape
  S = locs.shape[1]
  cpw = A // NW
  mesh = plsc.VectorSubcoreMesh(core_axis_name="c", subcore_axis_name="s")
  f = functools.partial(
      pl.kernel,
      out_type=jax.ShapeDtypeStruct((B, A), jnp.float32),
      mesh=mesh,
      compiler_params=pltpu.CompilerParams(needs_layout_passes=False),
      scratch_types=[
          pltpu.VMEM((cpw * S,), jnp.float32),
          pltpu.VMEM((cpw * S,), jnp.float32),
          pltpu.VMEM((ROWS, cpw), jnp.float32),
          pltpu.VMEM((ROWS, cpw), jnp.float32),
          pltpu.VMEM((ROWS, cpw), jnp.float32),
          pltpu.VMEM((ROWS, cpw), jnp.float32),
          pltpu.SemaphoreType.DMA,
          pltpu.SemaphoreType.DMA,
          pltpu.SemaphoreType.DMA,
          pltpu.SemaphoreType.DMA,
      ],
  )(_body)
  return f(x, locs.reshape(A * S), coeffs.reshape(A * S))
